# baseline (speedup 1.0000x reference)
import functools

import jax
import jax.numpy as jnp
from jax import lax
from jax.experimental import pallas as pl
from jax.experimental.pallas import tpu as pltpu
from jax.experimental.pallas import tpu_sc as plsc

_K = 20
_LAMDA = 0.003
_BIG = 1e30
_N = 1024
_L = 16
_NW = 32
_RPW = _N // _NW


def _gram_kernel(feats_ref, preds_ref, ycol_ref, g_ref, sq_ref, ce_ref):
    n = feats_ref.shape[0]
    f = feats_ref[:]
    g = jax.lax.dot_general(
        f, f, (((1,), (1,)), ((), ())),
        preferred_element_type=jnp.float32,
    )
    g_ref[:, :] = g

    rows = jax.lax.broadcasted_iota(jnp.int32, (n, n), 0)
    cols = jax.lax.broadcasted_iota(jnp.int32, (n, n), 1)
    sq_ref[:, :] = jnp.sum(jnp.where(rows == cols, g, 0.0),
                           axis=0, keepdims=True)

    p = preds_ref[:]
    c = preds_ref.shape[1]
    mx = jnp.max(p, axis=1, keepdims=True)
    lse = mx + jnp.log(jnp.sum(jnp.exp(p - mx), axis=1, keepdims=True))
    cls = jax.lax.broadcasted_iota(jnp.int32, (n, c), 1)
    sel = jnp.sum(jnp.where(cls == ycol_ref[:], p, 0.0),
                  axis=1, keepdims=True)
    ce_ref[0, 0] = jnp.sum(lse - sel) / n


def _sc_select_kernel(g_hbm, sq_hbm, y_hbm, a_hbm,
                      grows_v, key_v, grp_v, sq_v, y_v, sel_v, arows_v):
    wid = lax.axis_index("s") * 2 + lax.axis_index("c")
    base = wid * _RPW

    pltpu.sync_copy(g_hbm.at[pl.ds(base * _N, _RPW * _N)], grows_v)
    pltpu.sync_copy(sq_hbm, sq_v)
    pltpu.sync_copy(y_hbm, y_v)

    lane = lax.iota(jnp.int32, _L)
    lane0 = lane == 0
    lane4 = lane < 4
    onesf = jnp.broadcast_to(jnp.float32(1.0), (_L,))
    zerosf = jnp.broadcast_to(jnp.float32(0.0), (_L,))
    intmax = jnp.broadcast_to(jnp.int32(2**31 - 1), (_L,))
    nmask = jnp.int32(~1023)
    for c in range(_RPW * _N // _L):
        arows_v[pl.ds(c * _L, _L)] = zerosf

    def row_body(j, _):
        r = base + j
        r_vec = jnp.broadcast_to(r, (_L,))
        y_r = plsc.load_gather(y_v, [r_vec])
        sq_r = plsc.load_gather(sq_v, [r_vec])

        gmins = [intmax, intmax, intmax, intmax]
        for c in range(_N // _L):
            colv = lane + (c * _L)
            gv = grows_v[pl.ds(j * _N + c * _L, _L)]
            sqv = sq_v[pl.ds(c * _L, _L)]
            yv = y_v[pl.ds(c * _L, _L)]
            d2 = jnp.maximum(sq_r + sqv - 2.0 * gv, 0.0)
            md = jnp.where((yv == y_r) & (colv != r_vec), d2,
                           jnp.float32(_BIG))
            key = jnp.bitwise_or(
                jnp.bitwise_and(plsc.bitcast(md, jnp.int32), nmask), colv)
            key_v[pl.ds(c * _L, _L)] = key
            gmins[c // 16] = jnp.minimum(gmins[c // 16], key)
        for q in range(4):
            grp_v[pl.ds(q * _L, _L)] = gmins[q]

        def sel_body(t, carry):
            g0 = grp_v[pl.ds(0, _L)]
            g1 = grp_v[pl.ds(_L, _L)]
            g2 = grp_v[pl.ds(2 * _L, _L)]
            g3 = grp_v[pl.ds(3 * _L, _L)]
            m = jnp.minimum(jnp.minimum(g0, g1), jnp.minimum(g2, g3))
            kmin = lax.reduce_min(m, axes=(0,))
            col = jnp.bitwise_and(kmin, jnp.int32(1023))
            col_vec = jnp.broadcast_to(col, (_L,))
            plsc.store_scatter(sel_v, [jnp.broadcast_to(t, (_L,))],
                               col_vec, mask=lane0)
            plsc.store_scatter(key_v, [col_vec], intmax, mask=lane0)
            scan_base = jnp.bitwise_or(jnp.bitwise_and(col, jnp.int32(~255)),
                                       jnp.bitwise_and(col, jnp.int32(15)))
            scan_idx = jnp.broadcast_to(scan_base, (_L,)) + lane * _L
            newmin = lax.reduce_min(plsc.load_gather(key_v, [scan_idx]),
                                    axes=(0,))
            gpos = jnp.bitwise_or(
                jnp.right_shift(jnp.bitwise_and(col, jnp.int32(~255)),
                                jnp.int32(4)),
                jnp.bitwise_and(col, jnp.int32(15)))
            plsc.store_scatter(grp_v, [jnp.broadcast_to(gpos, (_L,))],
                               jnp.broadcast_to(newmin, (_L,)), mask=lane0)
            return carry

        lax.fori_loop(0, _K, sel_body, 0)

        sel_lo = sel_v[pl.ds(0, _L)] + (j * _N)
        sel_hi = sel_v[pl.ds(_L, _L)] + (j * _N)
        plsc.store_scatter(arows_v, [sel_lo], onesf)
        plsc.store_scatter(arows_v, [sel_hi], onesf, mask=lane4)
        return 0

    lax.fori_loop(0, _RPW, row_body, 0)
    pltpu.sync_copy(arows_v, a_hbm.at[pl.ds(base * _N, _RPW * _N)])


def _finish_kernel(a_ref, g_ref, sq_ref, ce_ref, out_ref):
    a = a_ref[:]
    g = g_ref[:]
    m = jax.lax.dot_general(
        a, g, (((1,), (0,)), ((), ())),
        preferred_element_type=jnp.float32,
    )
    s1 = jnp.sum(a * g)
    s2 = jnp.sum(a * m)
    kf = jnp.float32(_K)
    lp_sum = jnp.sum(sq_ref[:]) - (2.0 / kf) * s1 + s2 / (kf * kf)
    out_ref[0, 0] = (jnp.float32(_LAMDA) * (lp_sum / _N) / 2.0
                     + ce_ref[0, 0])


@jax.jit
def kernel(preds, feats, y):
    n = feats.shape[0]
    ycol = y.reshape(n, 1).astype(jnp.int32)
    g, sq, ce = pl.pallas_call(
        _gram_kernel,
        out_shape=(
            jax.ShapeDtypeStruct((n, n), jnp.float32),
            jax.ShapeDtypeStruct((1, n), jnp.float32),
            jax.ShapeDtypeStruct((1, 1), jnp.float32),
        ),
        out_specs=(
            pl.BlockSpec(memory_space=pltpu.VMEM),
            pl.BlockSpec(memory_space=pltpu.VMEM),
            pl.BlockSpec(memory_space=pltpu.SMEM),
        ),
    )(feats, preds, ycol)

    mesh = plsc.VectorSubcoreMesh(core_axis_name="c", subcore_axis_name="s")
    sc_fn = functools.partial(
        pl.kernel,
        mesh=mesh,
        out_type=jax.ShapeDtypeStruct((n * n,), jnp.float32),
        scratch_types=[
            pltpu.VMEM((_RPW * n,), jnp.float32),
            pltpu.VMEM((n,), jnp.int32),
            pltpu.VMEM((4 * _L,), jnp.int32),
            pltpu.VMEM((n,), jnp.float32),
            pltpu.VMEM((n,), jnp.int32),
            pltpu.VMEM((_NW,), jnp.int32),
            pltpu.VMEM((_RPW * n,), jnp.float32),
        ],
        compiler_params=pltpu.CompilerParams(needs_layout_passes=False),
    )(_sc_select_kernel)
    a = sc_fn(g.reshape(n * n), sq.reshape(n), y.astype(jnp.int32)).reshape(n, n)

    out = pl.pallas_call(
        _finish_kernel,
        out_shape=jax.ShapeDtypeStruct((1, 1), jnp.float32),
        in_specs=(
            pl.BlockSpec(memory_space=pltpu.VMEM),
            pl.BlockSpec(memory_space=pltpu.VMEM),
            pl.BlockSpec(memory_space=pltpu.VMEM),
            pl.BlockSpec(memory_space=pltpu.SMEM),
        ),
        out_specs=pl.BlockSpec(memory_space=pltpu.SMEM),
    )(a, g, sq, ce)
    return out[0, 0]


# speedup vs baseline: 1.0600x; 1.0600x over previous
import functools

import jax
import jax.numpy as jnp
from jax import lax
from jax.experimental import pallas as pl
from jax.experimental.pallas import tpu as pltpu
from jax.experimental.pallas import tpu_sc as plsc

_K = 20
_LAMDA = 0.003
_BIG = 1e30
_N = 1024
_L = 16
_NW = 32
_RPW = _N // _NW


def _gram_kernel(feats_ref, preds_ref, ycol_ref, g_ref, sq_ref, ce_ref):
    n = feats_ref.shape[0]
    f = feats_ref[:]
    g = jax.lax.dot_general(
        f, f, (((1,), (1,)), ((), ())),
        preferred_element_type=jnp.float32,
    )
    g_ref[:, :] = g

    rows = jax.lax.broadcasted_iota(jnp.int32, (n, n), 0)
    cols = jax.lax.broadcasted_iota(jnp.int32, (n, n), 1)
    sq_ref[:, :] = jnp.sum(jnp.where(rows == cols, g, 0.0),
                           axis=0, keepdims=True)

    p = preds_ref[:]
    c = preds_ref.shape[1]
    mx = jnp.max(p, axis=1, keepdims=True)
    lse = mx + jnp.log(jnp.sum(jnp.exp(p - mx), axis=1, keepdims=True))
    cls = jax.lax.broadcasted_iota(jnp.int32, (n, c), 1)
    sel = jnp.sum(jnp.where(cls == ycol_ref[:], p, 0.0),
                  axis=1, keepdims=True)
    ce_ref[0, 0] = jnp.sum(lse - sel) / n


def _sc_select_kernel(g_hbm, sq_hbm, y_hbm, a_hbm,
                      row_v, key_v, grp_v, sq_v, y_v, sel_v, arow_v):
    wid = lax.axis_index("s") * 2 + lax.axis_index("c")
    base = wid * _RPW

    pltpu.sync_copy(sq_hbm, sq_v)
    pltpu.sync_copy(y_hbm, y_v)

    lane = lax.iota(jnp.int32, _L)
    lane0 = lane == 0
    lane4 = lane < 4
    onesf = jnp.broadcast_to(jnp.float32(1.0), (_L,))
    zerosf = jnp.broadcast_to(jnp.float32(0.0), (_L,))
    negbig = jnp.broadcast_to(jnp.float32(-_BIG), (_L,))
    intmax = jnp.broadcast_to(jnp.int32(2**31 - 1), (_L,))
    nmask = jnp.int32(~1023)
    for c in range(_N // _L):
        arow_v[pl.ds(c * _L, _L)] = zerosf

    def row_body(j, _):
        r = base + j
        pltpu.sync_copy(g_hbm.at[r], row_v)
        r_vec = jnp.broadcast_to(r, (_L,))
        y_r = plsc.load_gather(y_v, [r_vec])
        sq_r = plsc.load_gather(sq_v, [r_vec])
        plsc.store_scatter(row_v, [r_vec], negbig, mask=lane0)

        gmins = [intmax, intmax, intmax, intmax]
        for c in range(_N // _L):
            colv = lane + (c * _L)
            gv = row_v[pl.ds(c * _L, _L)]
            sqv = sq_v[pl.ds(c * _L, _L)]
            yv = y_v[pl.ds(c * _L, _L)]
            d2 = jnp.maximum(sq_r + sqv - 2.0 * gv, 0.0)
            md = jnp.where(yv == y_r, d2, jnp.float32(_BIG))
            key = jnp.bitwise_or(
                jnp.bitwise_and(plsc.bitcast(md, jnp.int32), nmask), colv)
            key_v[pl.ds(c * _L, _L)] = key
            gmins[c // 16] = jnp.minimum(gmins[c // 16], key)
        for q in range(4):
            grp_v[pl.ds(q * _L, _L)] = gmins[q]

        def sel_body(t, carry):
            g0 = grp_v[pl.ds(0, _L)]
            g1 = grp_v[pl.ds(_L, _L)]
            g2 = grp_v[pl.ds(2 * _L, _L)]
            g3 = grp_v[pl.ds(3 * _L, _L)]
            m = jnp.minimum(jnp.minimum(g0, g1), jnp.minimum(g2, g3))
            kmin = lax.reduce_min(m, axes=(0,))
            col = jnp.bitwise_and(kmin, jnp.int32(1023))
            col_vec = jnp.broadcast_to(col, (_L,))
            plsc.store_scatter(sel_v, [jnp.broadcast_to(t, (_L,))],
                               col_vec, mask=lane0)
            plsc.store_scatter(key_v, [col_vec], intmax, mask=lane0)
            scan_base = jnp.bitwise_or(jnp.bitwise_and(col, jnp.int32(~255)),
                                       jnp.bitwise_and(col, jnp.int32(15)))
            scan_idx = jnp.broadcast_to(scan_base, (_L,)) + lane * _L
            newmin = lax.reduce_min(plsc.load_gather(key_v, [scan_idx]),
                                    axes=(0,))
            gpos = jnp.bitwise_or(
                jnp.right_shift(jnp.bitwise_and(col, jnp.int32(~255)),
                                jnp.int32(4)),
                jnp.bitwise_and(col, jnp.int32(15)))
            plsc.store_scatter(grp_v, [jnp.broadcast_to(gpos, (_L,))],
                               jnp.broadcast_to(newmin, (_L,)), mask=lane0)
            return carry

        lax.fori_loop(0, _K, sel_body, 0)

        sel_lo = sel_v[pl.ds(0, _L)]
        sel_hi = sel_v[pl.ds(_L, _L)]
        plsc.store_scatter(arow_v, [sel_lo], onesf)
        plsc.store_scatter(arow_v, [sel_hi], onesf, mask=lane4)
        pltpu.sync_copy(arow_v, a_hbm.at[r])
        plsc.store_scatter(arow_v, [sel_lo], zerosf)
        plsc.store_scatter(arow_v, [sel_hi], zerosf, mask=lane4)
        return 0

    lax.fori_loop(0, _RPW, row_body, 0)


def _finish_kernel(a_ref, g_ref, sq_ref, ce_ref, out_ref):
    a = a_ref[:]
    g = g_ref[:]
    m = jax.lax.dot_general(
        a.astype(jnp.bfloat16), g.astype(jnp.bfloat16),
        (((1,), (0,)), ((), ())),
        preferred_element_type=jnp.float32,
    )
    s1 = jnp.sum(a * g)
    s2 = jnp.sum(a * m)
    kf = jnp.float32(_K)
    lp_sum = jnp.sum(sq_ref[:]) - (2.0 / kf) * s1 + s2 / (kf * kf)
    out_ref[0, 0] = (jnp.float32(_LAMDA) * (lp_sum / _N) / 2.0
                     + ce_ref[0, 0])


@jax.jit
def kernel(preds, feats, y):
    n = feats.shape[0]
    ycol = y.reshape(n, 1).astype(jnp.int32)
    g, sq, ce = pl.pallas_call(
        _gram_kernel,
        out_shape=(
            jax.ShapeDtypeStruct((n, n), jnp.float32),
            jax.ShapeDtypeStruct((1, n), jnp.float32),
            jax.ShapeDtypeStruct((1, 1), jnp.float32),
        ),
        out_specs=(
            pl.BlockSpec(memory_space=pltpu.VMEM),
            pl.BlockSpec(memory_space=pltpu.VMEM),
            pl.BlockSpec(memory_space=pltpu.SMEM),
        ),
    )(feats.astype(jnp.bfloat16), preds, ycol)

    mesh = plsc.VectorSubcoreMesh(core_axis_name="c", subcore_axis_name="s")
    sc_fn = functools.partial(
        pl.kernel,
        mesh=mesh,
        out_type=jax.ShapeDtypeStruct((n, n), jnp.float32),
        scratch_types=[
            pltpu.VMEM((n,), jnp.float32),
            pltpu.VMEM((n,), jnp.int32),
            pltpu.VMEM((4 * _L,), jnp.int32),
            pltpu.VMEM((n,), jnp.float32),
            pltpu.VMEM((n,), jnp.int32),
            pltpu.VMEM((_NW,), jnp.int32),
            pltpu.VMEM((n,), jnp.float32),
        ],
        compiler_params=pltpu.CompilerParams(needs_layout_passes=False),
    )(_sc_select_kernel)
    a = sc_fn(g, sq.reshape(n), y.astype(jnp.int32))

    out = pl.pallas_call(
        _finish_kernel,
        out_shape=jax.ShapeDtypeStruct((1, 1), jnp.float32),
        in_specs=(
            pl.BlockSpec(memory_space=pltpu.VMEM),
            pl.BlockSpec(memory_space=pltpu.VMEM),
            pl.BlockSpec(memory_space=pltpu.VMEM),
            pl.BlockSpec(memory_space=pltpu.SMEM),
        ),
        out_specs=pl.BlockSpec(memory_space=pltpu.SMEM),
    )(a, g, sq, ce)
    return out[0, 0]


# speedup vs baseline: 1.0625x; 1.0023x over previous
import functools

import jax
import jax.numpy as jnp
from jax import lax
from jax.experimental import pallas as pl
from jax.experimental.pallas import tpu as pltpu
from jax.experimental.pallas import tpu_sc as plsc

_K = 20
_LAMDA = 0.003
_BIG = 1e30
_N = 1024
_L = 16
_NW = 32
_RPW = _N // _NW


def _gram_kernel(feats_ref, preds_ref, ycol_ref, g_ref, sq_ref, ce_ref):
    n = feats_ref.shape[0]
    f = feats_ref[:]
    g = jax.lax.dot_general(
        f, f, (((1,), (1,)), ((), ())),
        preferred_element_type=jnp.float32,
    )
    g_ref[:, :] = g

    rows = jax.lax.broadcasted_iota(jnp.int32, (n, n), 0)
    cols = jax.lax.broadcasted_iota(jnp.int32, (n, n), 1)
    sq_ref[:, :] = jnp.sum(jnp.where(rows == cols, g, 0.0),
                           axis=0, keepdims=True)

    p = preds_ref[:]
    c = preds_ref.shape[1]
    mx = jnp.max(p, axis=1, keepdims=True)
    lse = mx + jnp.log(jnp.sum(jnp.exp(p - mx), axis=1, keepdims=True))
    cls = jax.lax.broadcasted_iota(jnp.int32, (n, c), 1)
    sel = jnp.sum(jnp.where(cls == ycol_ref[:], p, 0.0),
                  axis=1, keepdims=True)
    ce_ref[0, 0] = jnp.sum(lse - sel) / n


def _sc_select_kernel(g_hbm, sq_hbm, y_hbm, a_hbm,
                      row_v, key_v, grp_v, sq_v, y_v, sel_v, arow_v):
    wid = lax.axis_index("s") * 2 + lax.axis_index("c")
    base = wid * _RPW

    pltpu.sync_copy(sq_hbm, sq_v)
    pltpu.sync_copy(y_hbm, y_v)

    lane = lax.iota(jnp.int32, _L)
    lane0 = lane == 0
    lane4 = lane < 4
    onesf = jnp.broadcast_to(jnp.float32(1.0), (_L,))
    zerosf = jnp.broadcast_to(jnp.float32(0.0), (_L,))
    negbig = jnp.broadcast_to(jnp.float32(-_BIG), (_L,))
    intmax = jnp.broadcast_to(jnp.int32(2**31 - 1), (_L,))
    nmask = jnp.int32(~1023)
    for c in range(_N // _L):
        arow_v[pl.ds(c * _L, _L)] = zerosf

    def pair_body(jj, _):
        pltpu.sync_copy(g_hbm.at[pl.ds((base + 2 * jj) * _N, 2 * _N)], row_v)
        r0 = base + 2 * jj
        r0_vec = jnp.broadcast_to(r0, (_L,))
        r1_vec = r0_vec + 1
        y_r0 = plsc.load_gather(y_v, [r0_vec])
        y_r1 = plsc.load_gather(y_v, [r1_vec])
        sq_r0 = plsc.load_gather(sq_v, [r0_vec])
        sq_r1 = plsc.load_gather(sq_v, [r1_vec])
        plsc.store_scatter(row_v, [r0_vec], negbig, mask=lane0)
        plsc.store_scatter(row_v, [r0_vec + (_N + 1)], negbig, mask=lane0)

        gmins0 = [intmax, intmax, intmax, intmax]
        gmins1 = [intmax, intmax, intmax, intmax]
        for c in range(_N // _L):
            colv = lane + (c * _L)
            sqv = sq_v[pl.ds(c * _L, _L)]
            yv = y_v[pl.ds(c * _L, _L)]
            gv0 = row_v[pl.ds(c * _L, _L)]
            gv1 = row_v[pl.ds(_N + c * _L, _L)]
            d20 = jnp.maximum(sq_r0 + sqv - 2.0 * gv0, 0.0)
            d21 = jnp.maximum(sq_r1 + sqv - 2.0 * gv1, 0.0)
            md0 = jnp.where(yv == y_r0, d20, jnp.float32(_BIG))
            md1 = jnp.where(yv == y_r1, d21, jnp.float32(_BIG))
            key0 = jnp.bitwise_or(
                jnp.bitwise_and(plsc.bitcast(md0, jnp.int32), nmask), colv)
            key1 = jnp.bitwise_or(
                jnp.bitwise_and(plsc.bitcast(md1, jnp.int32), nmask), colv)
            key_v[pl.ds(c * _L, _L)] = key0
            key_v[pl.ds(_N + c * _L, _L)] = key1
            gmins0[c // 16] = jnp.minimum(gmins0[c // 16], key0)
            gmins1[c // 16] = jnp.minimum(gmins1[c // 16], key1)

        def sel_and_emit(half, r, gmins):
            koff = half * _N
            koff_vec = jnp.broadcast_to(jnp.int32(koff), (_L,))
            for q in range(4):
                grp_v[pl.ds(q * _L, _L)] = gmins[q]

            def sel_body(t, carry):
                g0 = grp_v[pl.ds(0, _L)]
                g1 = grp_v[pl.ds(_L, _L)]
                g2 = grp_v[pl.ds(2 * _L, _L)]
                g3 = grp_v[pl.ds(3 * _L, _L)]
                m = jnp.minimum(jnp.minimum(g0, g1), jnp.minimum(g2, g3))
                kmin = lax.reduce_min(m, axes=(0,))
                col = jnp.bitwise_and(kmin, jnp.int32(1023))
                col_vec = jnp.broadcast_to(col, (_L,))
                plsc.store_scatter(sel_v, [jnp.broadcast_to(t, (_L,))],
                                   col_vec, mask=lane0)
                plsc.store_scatter(key_v, [col_vec + koff_vec], intmax,
                                   mask=lane0)
                scan_base = jnp.bitwise_or(
                    jnp.bitwise_and(col, jnp.int32(~255)),
                    jnp.bitwise_and(col, jnp.int32(15)))
                scan_idx = (jnp.broadcast_to(scan_base + koff, (_L,))
                            + lane * _L)
                newmin = lax.reduce_min(plsc.load_gather(key_v, [scan_idx]),
                                        axes=(0,))
                gpos = jnp.bitwise_or(
                    jnp.right_shift(jnp.bitwise_and(col, jnp.int32(~255)),
                                    jnp.int32(4)),
                    jnp.bitwise_and(col, jnp.int32(15)))
                plsc.store_scatter(grp_v, [jnp.broadcast_to(gpos, (_L,))],
                                   jnp.broadcast_to(newmin, (_L,)),
                                   mask=lane0)
                return carry

            lax.fori_loop(0, _K, sel_body, 0)

            sel_lo = sel_v[pl.ds(0, _L)]
            sel_hi = sel_v[pl.ds(_L, _L)]
            plsc.store_scatter(arow_v, [sel_lo], onesf)
            plsc.store_scatter(arow_v, [sel_hi], onesf, mask=lane4)
            pltpu.sync_copy(arow_v, a_hbm.at[r])
            plsc.store_scatter(arow_v, [sel_lo], zerosf)
            plsc.store_scatter(arow_v, [sel_hi], zerosf, mask=lane4)

        sel_and_emit(0, r0, gmins0)
        sel_and_emit(1, r0 + 1, gmins1)
        return 0

    lax.fori_loop(0, _RPW // 2, pair_body, 0)


def _finish_kernel(a_ref, g_ref, sq_ref, ce_ref, out_ref):
    a = a_ref[:]
    g = g_ref[:]
    m = jax.lax.dot_general(
        a.astype(jnp.bfloat16), g.astype(jnp.bfloat16),
        (((1,), (0,)), ((), ())),
        preferred_element_type=jnp.float32,
    )
    s1 = jnp.sum(a * g)
    s2 = jnp.sum(a * m)
    kf = jnp.float32(_K)
    lp_sum = jnp.sum(sq_ref[:]) - (2.0 / kf) * s1 + s2 / (kf * kf)
    out_ref[0, 0] = (jnp.float32(_LAMDA) * (lp_sum / _N) / 2.0
                     + ce_ref[0, 0])


@jax.jit
def kernel(preds, feats, y):
    n = feats.shape[0]
    ycol = y.reshape(n, 1).astype(jnp.int32)
    g, sq, ce = pl.pallas_call(
        _gram_kernel,
        out_shape=(
            jax.ShapeDtypeStruct((n, n), jnp.float32),
            jax.ShapeDtypeStruct((1, n), jnp.float32),
            jax.ShapeDtypeStruct((1, 1), jnp.float32),
        ),
        out_specs=(
            pl.BlockSpec(memory_space=pltpu.VMEM),
            pl.BlockSpec(memory_space=pltpu.VMEM),
            pl.BlockSpec(memory_space=pltpu.SMEM),
        ),
    )(feats.astype(jnp.bfloat16), preds, ycol)

    mesh = plsc.VectorSubcoreMesh(core_axis_name="c", subcore_axis_name="s")
    sc_fn = functools.partial(
        pl.kernel,
        mesh=mesh,
        out_type=jax.ShapeDtypeStruct((n, n), jnp.float32),
        scratch_types=[
            pltpu.VMEM((2 * n,), jnp.float32),
            pltpu.VMEM((2 * n,), jnp.int32),
            pltpu.VMEM((4 * _L,), jnp.int32),
            pltpu.VMEM((n,), jnp.float32),
            pltpu.VMEM((n,), jnp.int32),
            pltpu.VMEM((_NW,), jnp.int32),
            pltpu.VMEM((n,), jnp.float32),
        ],
        compiler_params=pltpu.CompilerParams(needs_layout_passes=False),
    )(_sc_select_kernel)
    a = sc_fn(g.reshape(n * n), sq.reshape(n), y.astype(jnp.int32))

    out = pl.pallas_call(
        _finish_kernel,
        out_shape=jax.ShapeDtypeStruct((1, 1), jnp.float32),
        in_specs=(
            pl.BlockSpec(memory_space=pltpu.VMEM),
            pl.BlockSpec(memory_space=pltpu.VMEM),
            pl.BlockSpec(memory_space=pltpu.VMEM),
            pl.BlockSpec(memory_space=pltpu.SMEM),
        ),
        out_specs=pl.BlockSpec(memory_space=pltpu.SMEM),
    )(a, g, sq, ce)
    return out[0, 0]


# speedup vs baseline: 1.2618x; 1.1876x over previous
import functools

import jax
import jax.numpy as jnp
from jax import lax
from jax.experimental import pallas as pl
from jax.experimental.pallas import tpu as pltpu
from jax.experimental.pallas import tpu_sc as plsc

_K = 20
_LAMDA = 0.003
_BIG = 1e30
_N = 1024
_L = 16
_NW = 32
_RPW = _N // _NW


def _gram_kernel(feats_ref, preds_ref, ycol_ref, g_ref, sq_ref, ce_ref):
    n = feats_ref.shape[0]
    f = feats_ref[:]
    g = jax.lax.dot_general(
        f, f, (((1,), (1,)), ((), ())),
        preferred_element_type=jnp.float32,
    )
    g_ref[:, :] = g

    rows = jax.lax.broadcasted_iota(jnp.int32, (n, n), 0)
    cols = jax.lax.broadcasted_iota(jnp.int32, (n, n), 1)
    sq_ref[:, :] = jnp.sum(jnp.where(rows == cols, g, 0.0),
                           axis=0, keepdims=True)

    p = preds_ref[:]
    c = preds_ref.shape[1]
    mx = jnp.max(p, axis=1, keepdims=True)
    lse = mx + jnp.log(jnp.sum(jnp.exp(p - mx), axis=1, keepdims=True))
    cls = jax.lax.broadcasted_iota(jnp.int32, (n, c), 1)
    sel = jnp.sum(jnp.where(cls == ycol_ref[:], p, 0.0),
                  axis=1, keepdims=True)
    ce_ref[0, 0] = jnp.sum(lse - sel) / n


def _sc_select_kernel(g_hbm, sq_hbm, y_hbm, a_hbm,
                      row_v, key_v, grp_v, sq_v, y_v, sel_v, arow_v):
    wid = lax.axis_index("s") * 2 + lax.axis_index("c")
    base = wid * _RPW

    pltpu.sync_copy(sq_hbm, sq_v)
    pltpu.sync_copy(y_hbm, y_v)

    lane = lax.iota(jnp.int32, _L)
    lane0 = lane == 0
    lane4 = lane < 4
    onesf = jnp.broadcast_to(jnp.float32(1.0), (_L,))
    zerosf = jnp.broadcast_to(jnp.float32(0.0), (_L,))
    negbig = jnp.broadcast_to(jnp.float32(-_BIG), (_L,))
    intmax = jnp.broadcast_to(jnp.int32(2**31 - 1), (_L,))
    nmask = jnp.int32(~1023)
    for c in range(_N // _L):
        arow_v[pl.ds(c * _L, _L)] = zerosf

    def pair_body(jj, _):
        pltpu.sync_copy(g_hbm.at[pl.ds((base + 2 * jj) * _N, 2 * _N)], row_v)
        r0 = base + 2 * jj
        r0_vec = jnp.broadcast_to(r0, (_L,))
        r1_vec = r0_vec + 1
        y_r0 = plsc.load_gather(y_v, [r0_vec])
        y_r1 = plsc.load_gather(y_v, [r1_vec])
        sq_r0 = plsc.load_gather(sq_v, [r0_vec])
        sq_r1 = plsc.load_gather(sq_v, [r1_vec])
        plsc.store_scatter(row_v, [r0_vec], negbig, mask=lane0)
        plsc.store_scatter(row_v, [r0_vec + (_N + 1)], negbig, mask=lane0)

        gmins0 = [intmax, intmax, intmax, intmax]
        gmins1 = [intmax, intmax, intmax, intmax]
        for c in range(_N // _L):
            colv = lane + (c * _L)
            sqv = sq_v[pl.ds(c * _L, _L)]
            yv = y_v[pl.ds(c * _L, _L)]
            gv0 = row_v[pl.ds(c * _L, _L)]
            gv1 = row_v[pl.ds(_N + c * _L, _L)]
            d20 = jnp.maximum(sq_r0 + sqv - 2.0 * gv0, 0.0)
            d21 = jnp.maximum(sq_r1 + sqv - 2.0 * gv1, 0.0)
            md0 = jnp.where(yv == y_r0, d20, jnp.float32(_BIG))
            md1 = jnp.where(yv == y_r1, d21, jnp.float32(_BIG))
            key0 = jnp.bitwise_or(
                jnp.bitwise_and(plsc.bitcast(md0, jnp.int32), nmask), colv)
            key1 = jnp.bitwise_or(
                jnp.bitwise_and(plsc.bitcast(md1, jnp.int32), nmask), colv)
            key_v[pl.ds(c * _L, _L)] = key0
            key_v[pl.ds(_N + c * _L, _L)] = key1
            gmins0[c // 16] = jnp.minimum(gmins0[c // 16], key0)
            gmins1[c // 16] = jnp.minimum(gmins1[c // 16], key1)

        for q in range(4):
            grp_v[pl.ds(q * _L, _L)] = gmins0[q]
            grp_v[pl.ds((4 + q) * _L, _L)] = gmins1[q]

        noff = jnp.broadcast_to(jnp.int32(_N), (_L,))

        def one_select(t, koff_vec, soff, goff, g0, g1, g2, g3):
            m = jnp.minimum(jnp.minimum(g0, g1), jnp.minimum(g2, g3))
            kmin = lax.reduce_min(m, axes=(0,))
            col = jnp.bitwise_and(kmin, jnp.int32(1023))
            col_vec = jnp.broadcast_to(col, (_L,))
            plsc.store_scatter(sel_v, [jnp.broadcast_to(t + soff, (_L,))],
                               col_vec, mask=lane0)
            plsc.store_scatter(key_v, [col_vec + koff_vec], intmax,
                               mask=lane0)
            scan_base = jnp.bitwise_or(
                jnp.bitwise_and(col, jnp.int32(~255)),
                jnp.bitwise_and(col, jnp.int32(15)))
            scan_idx = (jnp.broadcast_to(scan_base, (_L,)) + koff_vec
                        + lane * _L)
            newmin = lax.reduce_min(plsc.load_gather(key_v, [scan_idx]),
                                    axes=(0,))
            gpos = jnp.bitwise_or(
                jnp.right_shift(jnp.bitwise_and(col, jnp.int32(~255)),
                                jnp.int32(4)),
                jnp.bitwise_and(col, jnp.int32(15)))
            return gpos + goff, jnp.broadcast_to(newmin, (_L,))

        def sel_body(t, carry):
            ga0 = grp_v[pl.ds(0, _L)]
            ga1 = grp_v[pl.ds(_L, _L)]
            ga2 = grp_v[pl.ds(2 * _L, _L)]
            ga3 = grp_v[pl.ds(3 * _L, _L)]
            gb0 = grp_v[pl.ds(4 * _L, _L)]
            gb1 = grp_v[pl.ds(5 * _L, _L)]
            gb2 = grp_v[pl.ds(6 * _L, _L)]
            gb3 = grp_v[pl.ds(7 * _L, _L)]
            pa, va = one_select(t, noff * 0, 0, 0, ga0, ga1, ga2, ga3)
            pb, vb = one_select(t, noff, _NW, 4 * _L, gb0, gb1, gb2, gb3)
            plsc.store_scatter(grp_v, [jnp.broadcast_to(pa, (_L,))], va,
                               mask=lane0)
            plsc.store_scatter(grp_v, [jnp.broadcast_to(pb, (_L,))], vb,
                               mask=lane0)
            return carry

        lax.fori_loop(0, _K, sel_body, 0)

        for half, r in ((0, r0), (1, r0 + 1)):
            sel_lo = sel_v[pl.ds(half * _NW, _L)]
            sel_hi = sel_v[pl.ds(half * _NW + _L, _L)]
            plsc.store_scatter(arow_v, [sel_lo], onesf)
            plsc.store_scatter(arow_v, [sel_hi], onesf, mask=lane4)
            pltpu.sync_copy(arow_v, a_hbm.at[r])
            plsc.store_scatter(arow_v, [sel_lo], zerosf)
            plsc.store_scatter(arow_v, [sel_hi], zerosf, mask=lane4)
        return 0

    lax.fori_loop(0, _RPW // 2, pair_body, 0)


def _finish_kernel(a_ref, g_ref, sq_ref, ce_ref, out_ref):
    a = a_ref[:]
    g = g_ref[:]
    m = jax.lax.dot_general(
        a.astype(jnp.bfloat16), g.astype(jnp.bfloat16),
        (((1,), (0,)), ((), ())),
        preferred_element_type=jnp.float32,
    )
    s1 = jnp.sum(a * g)
    s2 = jnp.sum(a * m)
    kf = jnp.float32(_K)
    lp_sum = jnp.sum(sq_ref[:]) - (2.0 / kf) * s1 + s2 / (kf * kf)
    out_ref[0, 0] = (jnp.float32(_LAMDA) * (lp_sum / _N) / 2.0
                     + ce_ref[0, 0])


@jax.jit
def kernel(preds, feats, y):
    n = feats.shape[0]
    ycol = y.reshape(n, 1).astype(jnp.int32)
    g, sq, ce = pl.pallas_call(
        _gram_kernel,
        out_shape=(
            jax.ShapeDtypeStruct((n, n), jnp.float32),
            jax.ShapeDtypeStruct((1, n), jnp.float32),
            jax.ShapeDtypeStruct((1, 1), jnp.float32),
        ),
        out_specs=(
            pl.BlockSpec(memory_space=pltpu.VMEM),
            pl.BlockSpec(memory_space=pltpu.VMEM),
            pl.BlockSpec(memory_space=pltpu.SMEM),
        ),
    )(feats.astype(jnp.bfloat16), preds, ycol)

    mesh = plsc.VectorSubcoreMesh(core_axis_name="c", subcore_axis_name="s")
    sc_fn = functools.partial(
        pl.kernel,
        mesh=mesh,
        out_type=jax.ShapeDtypeStruct((n, n), jnp.float32),
        scratch_types=[
            pltpu.VMEM((2 * n,), jnp.float32),
            pltpu.VMEM((2 * n,), jnp.int32),
            pltpu.VMEM((8 * _L,), jnp.int32),
            pltpu.VMEM((n,), jnp.float32),
            pltpu.VMEM((n,), jnp.int32),
            pltpu.VMEM((2 * _NW,), jnp.int32),
            pltpu.VMEM((n,), jnp.float32),
        ],
        compiler_params=pltpu.CompilerParams(needs_layout_passes=False),
    )(_sc_select_kernel)
    a = sc_fn(g.reshape(n * n), sq.reshape(n), y.astype(jnp.int32))

    out = pl.pallas_call(
        _finish_kernel,
        out_shape=jax.ShapeDtypeStruct((1, 1), jnp.float32),
        in_specs=(
            pl.BlockSpec(memory_space=pltpu.VMEM),
            pl.BlockSpec(memory_space=pltpu.VMEM),
            pl.BlockSpec(memory_space=pltpu.VMEM),
            pl.BlockSpec(memory_space=pltpu.SMEM),
        ),
        out_specs=pl.BlockSpec(memory_space=pltpu.SMEM),
    )(a, g, sq, ce)
    return out[0, 0]
